# baseline (speedup 1.0000x reference)
import functools

import jax
import jax.numpy as jnp
from jax import lax
from jax.experimental import pallas as pl
from jax.experimental.pallas import tpu as pltpu
from jax.experimental.pallas import tpu_sc as plsc

NV = 1000000
D = 32
NB = 4096
NSQ = 200
NC, NS, LANES = 2, 16, 16
NW = NC * NS
VT = (NV + 127) // 128
VT_LAST = VT - 1
TAIL_V0 = VT_LAST * 128
TAIL_N = NV - TAIL_V0


def _tanh16(x):
    e = jnp.exp(x + x)
    return 1.0 - 2.0 / (e + 1.0)


_IOTA = lambda: lax.iota(jnp.int32, 16)


@functools.partial(
    pl.kernel,
    out_type=jax.ShapeDtypeStruct((NV * D,), jnp.float32),
    mesh=plsc.VectorSubcoreMesh(core_axis_name="c", subcore_axis_name="s"),
    scratch_types=[
        pltpu.VMEM((4, 8, 128), jnp.float32),
        pltpu.VMEM((128 * D,), jnp.float32),
    ],
    compiler_params=pltpu.CompilerParams(
        use_tc_tiling_on_sc=True, needs_layout_passes=False
    ),
)
def _retile(tbl_t_hbm, flat_hbm, blk_v, out_v):
    wid = lax.axis_index("s") * NC + lax.axis_index("c")
    cnt = (VT_LAST - wid + NW - 1) // NW

    def step(k, carry):
        t = wid + k * NW
        for r in range(4):
            pltpu.sync_copy(
                tbl_t_hbm.at[pl.ds(r * 8, 8), pl.ds(t * 128, 128)],
                blk_v.at[r],
            )

        @plsc.parallel_loop(0, 128, step=1, unroll=4)
        def lane(l):
            for h in range(2):
                d = h * 16 + _IOTA()
                vals = plsc.load_gather(
                    blk_v, [d >> 3, d & 7, jnp.full((16,), l, jnp.int32)]
                )
                out_v[pl.ds(l * D + h * 16, 16)] = vals

        pltpu.sync_copy(out_v, flat_hbm.at[pl.ds(t * 128 * D, 128 * D)])
        return carry

    lax.fori_loop(0, cnt, step, 0)


@functools.partial(
    pl.kernel,
    out_type=jax.ShapeDtypeStruct((NSQ, 4, NW, 1024), jnp.float32),
    mesh=plsc.VectorSubcoreMesh(core_axis_name="c", subcore_axis_name="s"),
    scratch_types=[
        pltpu.VMEM((128, NSQ), jnp.int32),
        pltpu.VMEM((NSQ, 128), jnp.int32),
        pltpu.VMEM((TAIL_N, D), jnp.float32),
        pltpu.VMEM((128, D), jnp.float32),
        pltpu.VMEM((4, 1024), jnp.float32),
        pltpu.SemaphoreType.DMA,
    ],
    compiler_params=pltpu.CompilerParams(
        use_tc_tiling_on_sc=False, needs_layout_passes=False
    ),
)
def _gather_tanh(
    ids_hbm, tbl_hbm, tail_hbm, out_hbm, idraw_v, idx_v, tail_v, rows_v, ob_v, sem
):
    wid = lax.axis_index("s") * NC + lax.axis_index("c")
    pltpu.sync_copy(tail_hbm, tail_v)
    pltpu.sync_copy(ids_hbm.at[pl.ds(wid * 128, 128)], idraw_v)

    @plsc.parallel_loop(0, NSQ, step=1, unroll=2)
    def tr_idx(s):
        for g in range(8):
            b = g * 16 + _IOTA()
            idx_v[s, pl.ds(g * 16, 16)] = plsc.load_gather(
                idraw_v, [b, jnp.full((16,), s, jnp.int32)]
            )

    def step(s, carry):
        pltpu.async_copy(tbl_hbm.at[idx_v.at[s]], rows_v, sem).wait()

        for g in range(8):
            bg = idx_v[s, pl.ds(g * 16, 16)]
            m = bg >= TAIL_V0
            adj = jnp.where(m, bg - TAIL_V0, 0)

            @plsc.parallel_loop(0, D, step=1, unroll=4)
            def cell(d):
                dv = jnp.full((16,), d, jnp.int32)
                vals = plsc.load_gather(rows_v, [g * 16 + _IOTA(), dv])
                tvals = plsc.load_gather(tail_v, [adj, dv])
                ob_v[d >> 3, pl.ds((d & 7) * 128 + g * 16, 16)] = _tanh16(
                    jnp.where(m, tvals, vals)
                )

        pltpu.sync_copy(ob_v, out_hbm.at[s, slice(None), wid])
        return carry

    lax.fori_loop(0, NSQ, step, 0)


def kernel(input_ids, table):
    flat = _retile(table.T)
    tbl = flat.reshape(NV, D)
    out4 = _gather_tanh(input_ids, tbl, table[TAIL_V0:])
    return (
        out4.reshape(NSQ, 4, NW, 8, 128)
        .transpose(2, 4, 0, 1, 3)
        .reshape(NB, NSQ, D)
    )


# speedup vs baseline: 1.8769x; 1.8769x over previous
import functools

import jax
import jax.numpy as jnp
from jax import lax
from jax.experimental import pallas as pl
from jax.experimental.pallas import tpu as pltpu
from jax.experimental.pallas import tpu_sc as plsc

NV = 1000000
D = 32
NB = 4096
NSQ = 200
NC, NS, LANES = 2, 16, 16
NW = NC * NS
VT = (NV + 127) // 128
VT_LAST = VT - 1
TAIL_V0 = VT_LAST * 128
TAIL_N = NV - TAIL_V0
GW = 512
NG = VT_LAST * 128 // GW


def _tanh16(x):
    e = jnp.exp(x + x)
    return 1.0 - 2.0 / (e + 1.0)


def _iota16():
    return lax.iota(jnp.int32, 16)


def _splat(v):
    return jnp.full((16,), v, jnp.int32)


@functools.partial(
    pl.kernel,
    out_type=jax.ShapeDtypeStruct((NV * D,), jnp.float32),
    mesh=plsc.VectorSubcoreMesh(core_axis_name="c", subcore_axis_name="s"),
    scratch_types=[
        pltpu.VMEM((2, D, GW), jnp.float32),
        pltpu.VMEM((2, GW * D), jnp.float32),
        pltpu.SemaphoreType.DMA,
        pltpu.SemaphoreType.DMA,
    ],
    compiler_params=pltpu.CompilerParams(
        use_tc_tiling_on_sc=True, needs_layout_passes=False
    ),
)
def _retile(tbl_t_hbm, flat_hbm, blk_v, out_v, sem_in, sem_out):
    wid = lax.axis_index("s") * NC + lax.axis_index("c")
    cnt = (NG - wid + NW - 1) // NW

    def fire(k, buf):
        c0 = (wid + k * NW) * GW
        pltpu.async_copy(
            tbl_t_hbm.at[slice(None), pl.ds(c0, GW)], blk_v.at[buf], sem_in
        )

    fire(0, 0)

    def step(k, carry):
        buf = k & 1
        pltpu.make_async_copy(
            tbl_t_hbm.at[slice(None), pl.ds(0, GW)], blk_v.at[buf], sem_in
        ).wait()

        @pl.when(k + 1 < cnt)
        def _():
            fire(k + 1, 1 - buf)

        @pl.when(k >= 2)
        def _():
            pltpu.make_async_copy(
                out_v.at[buf], flat_hbm.at[pl.ds(0, GW * D)], sem_out
            ).wait()

        bufv = _splat(buf)

        @plsc.parallel_loop(0, GW, step=1, unroll=4)
        def lane(l):
            lv = _splat(l)
            for h in range(2):
                vals = plsc.load_gather(blk_v, [bufv, h * 16 + _iota16(), lv])
                out_v[buf, pl.ds(l * D + h * 16, 16)] = vals

        pltpu.async_copy(
            out_v.at[buf],
            flat_hbm.at[pl.ds((wid + k * NW) * GW * D, GW * D)],
            sem_out,
        )
        return carry

    lax.fori_loop(0, cnt, step, 0)
    for _ in range(2):
        pltpu.make_async_copy(
            out_v.at[0], flat_hbm.at[pl.ds(0, GW * D)], sem_out
        ).wait()


@functools.partial(
    pl.kernel,
    out_type=jax.ShapeDtypeStruct((NSQ, 4, NW, 1024), jnp.float32),
    mesh=plsc.VectorSubcoreMesh(core_axis_name="c", subcore_axis_name="s"),
    scratch_types=[
        pltpu.VMEM((128, NSQ), jnp.int32),
        pltpu.VMEM((NSQ, 128), jnp.int32),
        pltpu.VMEM((TAIL_N, D), jnp.float32),
        pltpu.VMEM((2, 128, D), jnp.float32),
        pltpu.VMEM((2, 4, 1024), jnp.float32),
        pltpu.SemaphoreType.DMA,
        pltpu.SemaphoreType.DMA,
    ],
    compiler_params=pltpu.CompilerParams(
        use_tc_tiling_on_sc=False, needs_layout_passes=False
    ),
)
def _gather_tanh(
    ids_hbm,
    tbl_hbm,
    tail_hbm,
    out_hbm,
    idraw_v,
    idx_v,
    tail_v,
    rows_v,
    ob_v,
    sem_g,
    sem_w,
):
    wid = lax.axis_index("s") * NC + lax.axis_index("c")
    pltpu.sync_copy(tail_hbm, tail_v)
    pltpu.sync_copy(ids_hbm.at[pl.ds(wid * 128, 128)], idraw_v)

    @plsc.parallel_loop(0, NSQ, step=1, unroll=2)
    def tr_idx(s):
        sv = _splat(s)
        for g in range(8):
            idx_v[s, pl.ds(g * 16, 16)] = plsc.load_gather(
                idraw_v, [g * 16 + _iota16(), sv]
            )

    def fire(s, buf):
        pltpu.async_copy(tbl_hbm.at[idx_v.at[s]], rows_v.at[buf], sem_g)

    fire(0, 0)

    def step(s, carry):
        buf = s & 1
        pltpu.make_async_copy(
            tbl_hbm.at[idx_v.at[s]], rows_v.at[buf], sem_g
        ).wait()

        @pl.when(s + 1 < NSQ)
        def _():
            fire(s + 1, 1 - buf)

        @pl.when(s >= 2)
        def _():
            pltpu.make_async_copy(
                ob_v.at[buf], out_hbm.at[0, slice(None), wid], sem_w
            ).wait()

        bufv = _splat(buf)
        mx = idx_v[s, pl.ds(0, 16)]
        for g in range(1, 8):
            mx = jnp.maximum(mx, idx_v[s, pl.ds(g * 16, 16)])
        no_tail = jnp.max(mx) < TAIL_V0

        @pl.when(no_tail)
        def _():
            for g in range(8):
                bv = g * 16 + _iota16()

                @plsc.parallel_loop(0, D, step=1, unroll=4)
                def cell(d):
                    vals = plsc.load_gather(rows_v, [bufv, bv, _splat(d)])
                    ob_v[buf, d >> 3, pl.ds((d & 7) * 128 + g * 16, 16)] = (
                        _tanh16(vals)
                    )

        @pl.when(jnp.logical_not(no_tail))
        def _():
            for g in range(8):
                bv = g * 16 + _iota16()
                bg = idx_v[s, pl.ds(g * 16, 16)]
                m = bg >= TAIL_V0
                adj = jnp.where(m, bg - TAIL_V0, 0)

                @plsc.parallel_loop(0, D, step=1, unroll=4)
                def cell(d):
                    dv = _splat(d)
                    vals = plsc.load_gather(rows_v, [bufv, bv, dv])
                    tvals = plsc.load_gather(tail_v, [adj, dv])
                    ob_v[buf, d >> 3, pl.ds((d & 7) * 128 + g * 16, 16)] = (
                        _tanh16(jnp.where(m, tvals, vals))
                    )

        pltpu.async_copy(
            ob_v.at[buf], out_hbm.at[s, slice(None), wid], sem_w
        )
        return carry

    lax.fori_loop(0, NSQ, step, 0)
    for _ in range(2):
        pltpu.make_async_copy(
            ob_v.at[0], out_hbm.at[0, slice(None), wid], sem_w
        ).wait()


def kernel(input_ids, table):
    flat = _retile(table.T)
    tbl = flat.reshape(NV, D)
    out4 = _gather_tanh(input_ids, tbl, table[TAIL_V0:])
    return (
        out4.reshape(NSQ, 4, NW, 8, 128)
        .transpose(2, 4, 0, 1, 3)
        .reshape(NB, NSQ, D)
    )


# speedup vs baseline: 1.9073x; 1.0162x over previous
import functools

import jax
import jax.numpy as jnp
from jax import lax
from jax.experimental import pallas as pl
from jax.experimental.pallas import tpu as pltpu
from jax.experimental.pallas import tpu_sc as plsc

NV = 1000000
D = 32
NB = 4096
NSQ = 200
NC, NS, LANES = 2, 16, 16
NW = NC * NS
VT = (NV + 127) // 128
VT_LAST = VT - 1
TAIL_V0 = VT_LAST * 128
TAIL_N = NV - TAIL_V0
GW = 512
NG = VT_LAST * 128 // GW


def _tanh16(x):
    e = jnp.exp(x + x)
    return 1.0 - 2.0 / (e + 1.0)


def _iota16():
    return lax.iota(jnp.int32, 16)


def _splat(v):
    return jnp.full((16,), v, jnp.int32)


@functools.partial(
    pl.kernel,
    out_type=jax.ShapeDtypeStruct((NV * D,), jnp.float32),
    mesh=plsc.VectorSubcoreMesh(core_axis_name="c", subcore_axis_name="s"),
    scratch_types=[
        pltpu.VMEM((2, D, GW), jnp.float32),
        pltpu.VMEM((2, GW * D), jnp.float32),
        pltpu.SemaphoreType.DMA,
        pltpu.SemaphoreType.DMA,
    ],
    compiler_params=pltpu.CompilerParams(
        use_tc_tiling_on_sc=True,
        needs_layout_passes=False,
        disable_bounds_checks=True,
    ),
)
def _retile(tbl_t_hbm, flat_hbm, blk_v, out_v, sem_in, sem_out):
    wid = lax.axis_index("s") * NC + lax.axis_index("c")
    cnt = (NG - wid + NW - 1) // NW

    def fire(k, buf):
        c0 = (wid + k * NW) * GW
        pltpu.async_copy(
            tbl_t_hbm.at[slice(None), pl.ds(c0, GW)], blk_v.at[buf], sem_in
        )

    fire(0, 0)

    def step(k, carry):
        buf = k & 1
        pltpu.make_async_copy(
            tbl_t_hbm.at[slice(None), pl.ds(0, GW)], blk_v.at[buf], sem_in
        ).wait()

        @pl.when(k + 1 < cnt)
        def _():
            fire(k + 1, 1 - buf)

        @pl.when(k >= 2)
        def _():
            pltpu.make_async_copy(
                out_v.at[buf], flat_hbm.at[pl.ds(0, GW * D)], sem_out
            ).wait()

        bufv = _splat(buf)

        @plsc.parallel_loop(0, GW, step=1, unroll=8)
        def lane(l):
            lv = _splat(l)
            for h in range(2):
                vals = plsc.load_gather(blk_v, [bufv, h * 16 + _iota16(), lv])
                out_v[buf, pl.ds(l * D + h * 16, 16)] = vals

        pltpu.async_copy(
            out_v.at[buf],
            flat_hbm.at[pl.ds((wid + k * NW) * GW * D, GW * D)],
            sem_out,
        )
        return carry

    lax.fori_loop(0, cnt, step, 0)
    for _ in range(2):
        pltpu.make_async_copy(
            out_v.at[0], flat_hbm.at[pl.ds(0, GW * D)], sem_out
        ).wait()


@functools.partial(
    pl.kernel,
    out_type=jax.ShapeDtypeStruct((NSQ, 4, NW, 1024), jnp.float32),
    mesh=plsc.VectorSubcoreMesh(core_axis_name="c", subcore_axis_name="s"),
    scratch_types=[
        pltpu.VMEM((128, NSQ), jnp.int32),
        pltpu.VMEM((NSQ, 128), jnp.int32),
        pltpu.VMEM((TAIL_N, D), jnp.float32),
        pltpu.VMEM((2, 128, D), jnp.float32),
        pltpu.VMEM((2, 4, 1024), jnp.float32),
        pltpu.SemaphoreType.DMA,
        pltpu.SemaphoreType.DMA,
    ],
    compiler_params=pltpu.CompilerParams(
        use_tc_tiling_on_sc=False,
        needs_layout_passes=False,
        disable_bounds_checks=True,
    ),
)
def _gather_tanh(
    ids_hbm,
    tbl_hbm,
    tail_hbm,
    out_hbm,
    idraw_v,
    idx_v,
    tail_v,
    rows_v,
    ob_v,
    sem_g,
    sem_w,
):
    wid = lax.axis_index("s") * NC + lax.axis_index("c")
    pltpu.sync_copy(tail_hbm, tail_v)
    pltpu.sync_copy(ids_hbm.at[pl.ds(wid * 128, 128)], idraw_v)

    @plsc.parallel_loop(0, NSQ, step=1, unroll=2)
    def tr_idx(s):
        sv = _splat(s)
        for g in range(8):
            idx_v[s, pl.ds(g * 16, 16)] = plsc.load_gather(
                idraw_v, [g * 16 + _iota16(), sv]
            )

    def fire(s, buf):
        pltpu.async_copy(tbl_hbm.at[idx_v.at[s]], rows_v.at[buf], sem_g)

    fire(0, 0)

    def step(s, carry):
        buf = s & 1
        pltpu.make_async_copy(
            tbl_hbm.at[idx_v.at[s]], rows_v.at[buf], sem_g
        ).wait()

        @pl.when(s + 1 < NSQ)
        def _():
            fire(s + 1, 1 - buf)

        @pl.when(s >= 2)
        def _():
            pltpu.make_async_copy(
                ob_v.at[buf], out_hbm.at[0, slice(None), wid], sem_w
            ).wait()

        bufv = _splat(buf)
        mx = idx_v[s, pl.ds(0, 16)]
        for g in range(1, 8):
            mx = jnp.maximum(mx, idx_v[s, pl.ds(g * 16, 16)])
        no_tail = jnp.max(mx) < TAIL_V0

        @pl.when(no_tail)
        def _():
            for g in range(8):
                bv = g * 16 + _iota16()

                @plsc.parallel_loop(0, D, step=1, unroll=8)
                def cell(d):
                    vals = plsc.load_gather(rows_v, [bufv, bv, _splat(d)])
                    ob_v[buf, d >> 3, pl.ds((d & 7) * 128 + g * 16, 16)] = (
                        _tanh16(vals)
                    )

        @pl.when(jnp.logical_not(no_tail))
        def _():
            for g in range(8):
                bv = g * 16 + _iota16()
                bg = idx_v[s, pl.ds(g * 16, 16)]
                m = bg >= TAIL_V0
                adj = jnp.where(m, bg - TAIL_V0, 0)

                @plsc.parallel_loop(0, D, step=1, unroll=8)
                def cell(d):
                    dv = _splat(d)
                    vals = plsc.load_gather(rows_v, [bufv, bv, dv])
                    tvals = plsc.load_gather(tail_v, [adj, dv])
                    ob_v[buf, d >> 3, pl.ds((d & 7) * 128 + g * 16, 16)] = (
                        _tanh16(jnp.where(m, tvals, vals))
                    )

        pltpu.async_copy(
            ob_v.at[buf], out_hbm.at[s, slice(None), wid], sem_w
        )
        return carry

    lax.fori_loop(0, NSQ, step, 0)
    for _ in range(2):
        pltpu.make_async_copy(
            ob_v.at[0], out_hbm.at[0, slice(None), wid], sem_w
        ).wait()


def kernel(input_ids, table):
    flat = _retile(table.T)
    tbl = flat.reshape(NV, D)
    out4 = _gather_tanh(input_ids, tbl, table[TAIL_V0:])
    return (
        out4.reshape(NSQ, 4, NW, 8, 128)
        .transpose(2, 4, 0, 1, 3)
        .reshape(NB, NSQ, D)
    )
